# Initial kernel scaffold; baseline (speedup 1.0000x reference)
import jax
import jax.numpy as jnp
from jax.experimental import pallas as pl


def kernel(z, pos, batch, edge_index, emb, W_dist, b_dist, Wn, We, be, Wo, bo, W1, b1, W2, b2):
    raise NotImplementedError("write your pallas kernel here")


# speedup vs baseline: 2.4384x; 2.4384x over previous
import functools

import jax
import jax.numpy as jnp
from jax import lax
from jax.experimental import pallas as pl
from jax.experimental.pallas import tpu as pltpu
from jax.experimental.pallas import tpu_sc as plsc

N = 10000
E = 320000
H = 128
NG = 50
L = 6
B = 64

NC = 2
NS = 16
NW = NC * NS

PCH = 2000
EPT = E // NW
VCH = 80
VWORKERS = 25
VCHUNKS = N // (VWORKERS * VCH)

K = 80
EPC = E // NC
EPT2 = EPC // NS
NCHUNK = EPT2 // K
RS = 624
RSREM = N - NS * RS

BN = 1000

_mesh = plsc.VectorSubcoreMesh(core_axis_name="c", subcore_axis_name="s")
_sc_params = pltpu.CompilerParams(needs_layout_passes=False)


@functools.partial(
    pl.kernel,
    out_type=(
        jax.ShapeDtypeStruct((E,), jnp.float32),
        jax.ShapeDtypeStruct((N, H), jnp.float32),
    ),
    mesh=_mesh,
    compiler_params=_sc_params,
    scratch_types=[
        pltpu.VMEM((N,), jnp.float32),
        pltpu.VMEM((N,), jnp.float32),
        pltpu.VMEM((N,), jnp.float32),
        pltpu.VMEM((PCH,), jnp.int32),
        pltpu.VMEM((PCH,), jnp.int32),
        pltpu.VMEM((PCH,), jnp.float32),
        pltpu.VMEM((VCH,), jnp.int32),
        pltpu.VMEM((VCH, H), jnp.float32),
        pltpu.SemaphoreType.DMA,
    ],
)
def _prep(px_hbm, py_hbm, pz_hbm, row_hbm, col_hbm, z_hbm, emb_hbm,
          d2_hbm, v_hbm,
          px, py, pz, rowb, colb, d2b, zb, erows, sem):
  c = lax.axis_index("c")
  s = lax.axis_index("s")
  wid = c * NS + s

  pltpu.sync_copy(px_hbm, px)
  pltpu.sync_copy(py_hbm, py)
  pltpu.sync_copy(pz_hbm, pz)

  ebase = wid * EPT

  def chunk_body(ch, carry):
    off = ebase + ch * PCH
    pltpu.sync_copy(row_hbm.at[pl.ds(off, PCH)], rowb)
    pltpu.sync_copy(col_hbm.at[pl.ds(off, PCH)], colb)

    def grp(g, carry2):
      r16 = rowb[pl.ds(g * 16, 16)]
      c16 = colb[pl.ds(g * 16, 16)]
      dx = plsc.load_gather(px, [r16]) - plsc.load_gather(px, [c16])
      dy = plsc.load_gather(py, [r16]) - plsc.load_gather(py, [c16])
      dz = plsc.load_gather(pz, [r16]) - plsc.load_gather(pz, [c16])
      d2b[pl.ds(g * 16, 16)] = dx * dx + dy * dy + dz * dz
      return carry2

    lax.fori_loop(0, PCH // 16, grp, 0)
    pltpu.sync_copy(d2b, d2_hbm.at[pl.ds(off, PCH)])
    return carry

  lax.fori_loop(0, EPT // PCH, chunk_body, 0)

  @pl.when(wid < VWORKERS)
  def _():
    def vchunk(k, carry):
      rbase = wid * (VCH * VCHUNKS) + k * VCH
      pltpu.sync_copy(z_hbm.at[pl.ds(rbase, VCH)], zb)
      pltpu.async_copy(emb_hbm.at[zb], erows, sem).wait()
      pltpu.sync_copy(erows, v_hbm.at[pl.ds(rbase, VCH)])
      return carry

    lax.fori_loop(0, VCHUNKS, vchunk, 0)


def _sqrt_body(d2_ref, o_ref):
  o_ref[...] = jnp.sqrt(d2_ref[...] + 1e-12)


def _dist_tc(d2):
  d2m = d2.reshape(E // 128, 128)
  out = pl.pallas_call(
      _sqrt_body,
      out_shape=jax.ShapeDtypeStruct((E // 128, 128), jnp.float32),
  )(d2m)
  return out.reshape(E, 1)


@functools.partial(
    pl.kernel,
    out_type=jax.ShapeDtypeStruct((NC, N, H), jnp.float32),
    mesh=_mesh,
    compiler_params=_sc_params,
    scratch_types=[
        pltpu.VMEM_SHARED((N, H), jnp.float32),
        pltpu.VMEM((K,), jnp.int32),
        pltpu.VMEM((K,), jnp.int32),
        pltpu.VMEM((K, H), jnp.float32),
        pltpu.VMEM((K, H), jnp.float32),
        pltpu.SemaphoreType.DMA,
    ],
)
def _accum(h_hbm, row_hbm, col_hbm, ew_hbm, zeros_hbm,
           agg_hbm,
           acc, rowb, colb, ewb, rows, sem):
  c = lax.axis_index("c")
  s = lax.axis_index("s")

  rs = s * RS
  pltpu.sync_copy(zeros_hbm.at[pl.ds(rs, RS)], acc.at[pl.ds(rs, RS)])

  @pl.when(s == NS - 1)
  def _():
    pltpu.sync_copy(zeros_hbm.at[pl.ds(NS * RS, RSREM)],
                    acc.at[pl.ds(NS * RS, RSREM)])

  plsc.subcore_barrier()

  ebase = c * EPC + s * EPT2

  def chunk(ch, carry):
    off = ebase + ch * K
    pltpu.sync_copy(row_hbm.at[pl.ds(off, K)], rowb)
    pltpu.sync_copy(col_hbm.at[pl.ds(off, K)], colb)
    pltpu.sync_copy(ew_hbm.at[pl.ds(off, K)], ewb)
    pltpu.async_copy(h_hbm.at[colb], rows, sem).wait()

    def grp(g, carry2):
      for e in range(16):
        idx = g * 16 + e
        for j in range(H // 16):
          rows[idx, pl.ds(j * 16, 16)] = (
              rows[idx, pl.ds(j * 16, 16)] * ewb[idx, pl.ds(j * 16, 16)])
      return carry2

    lax.fori_loop(0, K // 16, grp, 0)

    pltpu.sync_copy(rows, acc.at[rowb], add=True)
    return carry

  lax.fori_loop(0, NCHUNK, chunk, 0)
  plsc.subcore_barrier()

  pltpu.sync_copy(acc.at[pl.ds(rs, RS)], agg_hbm.at[c, pl.ds(rs, RS)])

  @pl.when(s == NS - 1)
  def _():
    pltpu.sync_copy(acc.at[pl.ds(NS * RS, RSREM)],
                    agg_hbm.at[c, pl.ds(NS * RS, RSREM)])


def _ssp_tc(x):
  return jnp.maximum(x, 0.0) + jnp.log1p(jnp.exp(-jnp.abs(x))) - 0.6931471805599453


def _dot_hi(x, w):
  return lax.dot_general(x, w, (((1,), (0,)), ((), ())),
                         precision=lax.Precision.HIGHEST,
                         preferred_element_type=jnp.float32)


def _dot_bf(x, w):
  return lax.dot_general(x.astype(jnp.bfloat16), w.astype(jnp.bfloat16),
                         (((1,), (0,)), ((), ())),
                         preferred_element_type=jnp.float32)


def _transform_body(v_ref, wn_ref, h_ref):
  h_ref[...] = _dot_bf(v_ref[...], wn_ref[...])


def _transform_tc(v, Wn_l):
  rowblk = pl.BlockSpec((BN, H), lambda i: (i, 0))
  return pl.pallas_call(
      _transform_body,
      grid=(N // BN,),
      in_specs=[rowblk, pl.BlockSpec((H, H), lambda i: (0, 0))],
      out_specs=rowblk,
      out_shape=jax.ShapeDtypeStruct((N, H), jnp.float32),
  )(v, Wn_l)


def _combine_body(agg_ref, wo_ref, bo_ref, v_ref):
  agg = agg_ref[0] + agg_ref[1]
  o = _dot_bf(agg, wo_ref[...]) + bo_ref[...]
  v_ref[...] = _ssp_tc(o)


def _combine_tc(AGG, Wo_l, bo_l):
  rowblk = pl.BlockSpec((BN, H), lambda i: (i, 0))
  full = lambda shape: pl.BlockSpec(shape, lambda i: tuple(0 for _ in shape))
  return pl.pallas_call(
      _combine_body,
      grid=(N // BN,),
      in_specs=[pl.BlockSpec((NC, BN, H), lambda i: (0, i, 0)),
                full((H, H)), full((1, H))],
      out_specs=rowblk,
      out_shape=jax.ShapeDtypeStruct((N, H), jnp.float32),
  )(AGG, Wo_l, bo_l)


BE = 3200


def _ew_body(dist_ref, wd_ref, bd_ref, we_ref, be_ref, ew_ref):
  e = _dot_hi(dist_ref[...], wd_ref[...]) + bd_ref[...]
  ew_ref[...] = _dot_bf(e, we_ref[...]) + be_ref[...]


def _ew_tc(dist_col, Wd, bd, We_l, be_l):
  full = lambda shape: pl.BlockSpec(shape, lambda i: tuple(0 for _ in shape))
  return pl.pallas_call(
      _ew_body,
      grid=(E // BE,),
      in_specs=[pl.BlockSpec((BE, 1), lambda i: (i, 0)),
                full((1, NG)), full((1, NG)), full((NG, H)), full((1, H))],
      out_specs=pl.BlockSpec((BE, H), lambda i: (i, 0)),
      out_shape=jax.ShapeDtypeStruct((E, H), jnp.float32),
  )(dist_col, Wd, bd, We_l, be_l)


def _readout_body(v_ref, bt_ref, w1_ref, b1_ref, w2_ref, b2_ref, u_ref):
  t1 = _ssp_tc(_dot_bf(v_ref[...], w1_ref[...]) + b1_ref[...])
  t = _dot_bf(t1, w2_ref[...]) + b2_ref[...]
  bvec = bt_ref[0, 0, :]
  oh = (bvec[:, None] == lax.broadcasted_iota(jnp.int32, (BN, B), 1))
  ub = lax.dot_general(oh.astype(jnp.float32), t, (((0,), (0,)), ((), ())),
                       precision=lax.Precision.HIGHEST,
                       preferred_element_type=jnp.float32)

  @pl.when(pl.program_id(0) == 0)
  def _():
    u_ref[...] = jnp.zeros_like(u_ref)

  u_ref[...] += ub


def _readout_tc(v, batch3, W1, b1, W2t, b2t):
  full = lambda shape: pl.BlockSpec(shape, lambda i: tuple(0 for _ in shape))
  return pl.pallas_call(
      _readout_body,
      grid=(N // BN,),
      in_specs=[pl.BlockSpec((BN, H), lambda i: (i, 0)),
                pl.BlockSpec((1, 1, BN), lambda i: (i, 0, 0)),
                full((H, H // 2)), full((1, H // 2)),
                full((H // 2, 128)), full((1, 128))],
      out_specs=full((B, 128)),
      out_shape=jax.ShapeDtypeStruct((B, 128), jnp.float32),
  )(v, batch3, W1, b1, W2t, b2t)


def kernel(z, pos, batch, edge_index, emb, W_dist, b_dist, Wn, We, be, Wo, bo,
           W1, b1, W2, b2):
  row = edge_index[0].astype(jnp.int32)
  col = edge_index[1].astype(jnp.int32)
  z32 = z.astype(jnp.int32)
  posx = pos[:, 0]
  posy = pos[:, 1]
  posz = pos[:, 2]
  zerosNH = jnp.zeros((N, H), jnp.float32)

  d2, v = _prep(posx, posy, posz, row, col, z32, emb)
  dist_col = _dist_tc(d2)

  Wd = W_dist.reshape(1, NG)
  bd = b_dist.reshape(1, NG)
  for l in range(L):
    ew = _ew_tc(dist_col, Wd, bd, We[l], be[l].reshape(1, H))
    h = _transform_tc(v, Wn[l])
    AGG = _accum(h, row, col, ew, zerosNH)
    v = _combine_tc(AGG, Wo[l], bo[l].reshape(1, H))

  batch3 = batch.astype(jnp.int32).reshape(N // BN, 1, BN)
  b1r = b1.reshape(1, H // 2)
  W2t = jnp.broadcast_to(W2, (H // 2, 128))
  b2t = jnp.broadcast_to(b2.reshape(1, 1), (1, 128))
  u128 = _readout_tc(v, batch3, W1, b1r, W2t, b2t)
  return u128[:, :1]


# speedup vs baseline: 3.6988x; 1.5169x over previous
import functools

import jax
import jax.numpy as jnp
from jax import lax
from jax.experimental import pallas as pl
from jax.experimental.pallas import tpu as pltpu
from jax.experimental.pallas import tpu_sc as plsc

N = 10000
E = 320000
H = 128
NG = 50
L = 6
B = 64

NC = 2
NS = 16
NW = NC * NS

PCH = 2000
EPT = E // NW
VCH = 80
VWORKERS = 25
VCHUNKS = N // (VWORKERS * VCH)

K = 80
EPC = E // NC
EPT2 = EPC // NS
NCHUNK = EPT2 // K
RS = 624
RSREM = N - NS * RS

BN = 1000

_mesh = plsc.VectorSubcoreMesh(core_axis_name="c", subcore_axis_name="s")
_sc_params = pltpu.CompilerParams(needs_layout_passes=False)


@functools.partial(
    pl.kernel,
    out_type=(
        jax.ShapeDtypeStruct((E,), jnp.float32),
        jax.ShapeDtypeStruct((N, H), jnp.float32),
    ),
    mesh=_mesh,
    compiler_params=_sc_params,
    scratch_types=[
        pltpu.VMEM((N,), jnp.float32),
        pltpu.VMEM((N,), jnp.float32),
        pltpu.VMEM((N,), jnp.float32),
        pltpu.VMEM((PCH,), jnp.int32),
        pltpu.VMEM((PCH,), jnp.int32),
        pltpu.VMEM((PCH,), jnp.float32),
        pltpu.VMEM((VCH,), jnp.int32),
        pltpu.VMEM((VCH, H), jnp.float32),
        pltpu.SemaphoreType.DMA,
    ],
)
def _prep(px_hbm, py_hbm, pz_hbm, row_hbm, col_hbm, z_hbm, emb_hbm,
          d2_hbm, v_hbm,
          px, py, pz, rowb, colb, d2b, zb, erows, sem):
  c = lax.axis_index("c")
  s = lax.axis_index("s")
  wid = c * NS + s

  pltpu.sync_copy(px_hbm, px)
  pltpu.sync_copy(py_hbm, py)
  pltpu.sync_copy(pz_hbm, pz)

  ebase = wid * EPT

  def chunk_body(ch, carry):
    off = ebase + ch * PCH
    pltpu.sync_copy(row_hbm.at[pl.ds(off, PCH)], rowb)
    pltpu.sync_copy(col_hbm.at[pl.ds(off, PCH)], colb)

    def grp(g, carry2):
      r16 = rowb[pl.ds(g * 16, 16)]
      c16 = colb[pl.ds(g * 16, 16)]
      dx = plsc.load_gather(px, [r16]) - plsc.load_gather(px, [c16])
      dy = plsc.load_gather(py, [r16]) - plsc.load_gather(py, [c16])
      dz = plsc.load_gather(pz, [r16]) - plsc.load_gather(pz, [c16])
      d2b[pl.ds(g * 16, 16)] = dx * dx + dy * dy + dz * dz
      return carry2

    lax.fori_loop(0, PCH // 16, grp, 0)
    pltpu.sync_copy(d2b, d2_hbm.at[pl.ds(off, PCH)])
    return carry

  lax.fori_loop(0, EPT // PCH, chunk_body, 0)

  @pl.when(wid < VWORKERS)
  def _():
    def vchunk(k, carry):
      rbase = wid * (VCH * VCHUNKS) + k * VCH
      pltpu.sync_copy(z_hbm.at[pl.ds(rbase, VCH)], zb)
      pltpu.async_copy(emb_hbm.at[zb], erows, sem).wait()
      pltpu.sync_copy(erows, v_hbm.at[pl.ds(rbase, VCH)])
      return carry

    lax.fori_loop(0, VCHUNKS, vchunk, 0)


def _sqrt_body(d2_ref, o_ref):
  o_ref[...] = jnp.sqrt(d2_ref[...] + 1e-12)


def _dist_tc(d2):
  d2m = d2.reshape(E // 128, 128)
  out = pl.pallas_call(
      _sqrt_body,
      out_shape=jax.ShapeDtypeStruct((E // 128, 128), jnp.float32),
  )(d2m)
  return out.reshape(E, 1)


@functools.partial(
    pl.kernel,
    out_type=jax.ShapeDtypeStruct((NC, N, H), jnp.float32),
    mesh=_mesh,
    compiler_params=_sc_params,
    scratch_types=[
        pltpu.VMEM_SHARED((N, H), jnp.float32),
        pltpu.VMEM((K,), jnp.int32),
        pltpu.VMEM((K,), jnp.int32),
        pltpu.VMEM((K, H), jnp.float32),
        pltpu.VMEM((K, H), jnp.float32),
        pltpu.VMEM((K,), jnp.int32),
        pltpu.VMEM((K,), jnp.int32),
        pltpu.VMEM((K, H), jnp.float32),
        pltpu.VMEM((K, H), jnp.float32),
        pltpu.SemaphoreType.DMA,
        pltpu.SemaphoreType.DMA,
        pltpu.SemaphoreType.DMA,
        pltpu.SemaphoreType.DMA,
        pltpu.SemaphoreType.DMA,
        pltpu.SemaphoreType.DMA,
    ],
)
def _accum(h_hbm, row_hbm, col_hbm, ew_hbm, zeros_hbm,
           agg_hbm,
           acc, rowbA, colbA, ewbA, rowsA, rowbB, colbB, ewbB, rowsB,
           semIA, semIB, semGA, semGB, semSA, semSB):
  c = lax.axis_index("c")
  s = lax.axis_index("s")

  rs = s * RS
  pltpu.sync_copy(zeros_hbm.at[pl.ds(rs, RS)], acc.at[pl.ds(rs, RS)])

  @pl.when(s == NS - 1)
  def _():
    pltpu.sync_copy(zeros_hbm.at[pl.ds(NS * RS, RSREM)],
                    acc.at[pl.ds(NS * RS, RSREM)])

  plsc.subcore_barrier()

  ebase = c * EPC + s * EPT2

  def _multiply(rows, ewb):
    def grp(g, carry2):
      for e in range(16):
        idx = g * 16 + e
        for j in range(H // 16):
          rows[idx, pl.ds(j * 16, 16)] = (
              rows[idx, pl.ds(j * 16, 16)] * ewb[idx, pl.ds(j * 16, 16)])
      return carry2

    lax.fori_loop(0, K // 16, grp, 0)

  def pair(i, carry):
    offA = ebase + (2 * i) * K
    offB = offA + K
    cA = pltpu.async_copy(col_hbm.at[pl.ds(offA, K)], colbA, semIA)
    rA = pltpu.async_copy(row_hbm.at[pl.ds(offA, K)], rowbA, semIA)
    cB = pltpu.async_copy(col_hbm.at[pl.ds(offB, K)], colbB, semIB)
    rB = pltpu.async_copy(row_hbm.at[pl.ds(offB, K)], rowbB, semIB)
    eA = pltpu.async_copy(ew_hbm.at[pl.ds(offA, K)], ewbA, semGA)
    eB = pltpu.async_copy(ew_hbm.at[pl.ds(offB, K)], ewbB, semGB)
    cA.wait()
    gA = pltpu.async_copy(h_hbm.at[colbA], rowsA, semGA)
    cB.wait()
    gB = pltpu.async_copy(h_hbm.at[colbB], rowsB, semGB)
    gA.wait()
    eA.wait()
    _multiply(rowsA, ewbA)
    rA.wait()
    sA = pltpu.async_copy(rowsA, acc.at[rowbA], semSA, add=True)
    gB.wait()
    eB.wait()
    _multiply(rowsB, ewbB)
    rB.wait()
    sB = pltpu.async_copy(rowsB, acc.at[rowbB], semSB, add=True)
    sA.wait()
    sB.wait()
    return carry

  lax.fori_loop(0, NCHUNK // 2, pair, 0)

  off = ebase + (NCHUNK - 1) * K
  pltpu.sync_copy(row_hbm.at[pl.ds(off, K)], rowbA)
  pltpu.sync_copy(col_hbm.at[pl.ds(off, K)], colbA)
  pltpu.sync_copy(ew_hbm.at[pl.ds(off, K)], ewbA)
  pltpu.async_copy(h_hbm.at[colbA], rowsA, semGA).wait()
  _multiply(rowsA, ewbA)
  pltpu.sync_copy(rowsA, acc.at[rowbA], add=True)
  plsc.subcore_barrier()

  pltpu.sync_copy(acc.at[pl.ds(rs, RS)], agg_hbm.at[c, pl.ds(rs, RS)])

  @pl.when(s == NS - 1)
  def _():
    pltpu.sync_copy(acc.at[pl.ds(NS * RS, RSREM)],
                    agg_hbm.at[c, pl.ds(NS * RS, RSREM)])


def _ssp_tc(x):
  return jnp.maximum(x, 0.0) + jnp.log1p(jnp.exp(-jnp.abs(x))) - 0.6931471805599453


def _dot_hi(x, w):
  return lax.dot_general(x, w, (((1,), (0,)), ((), ())),
                         precision=lax.Precision.HIGHEST,
                         preferred_element_type=jnp.float32)


def _dot_bf(x, w):
  return lax.dot_general(x.astype(jnp.bfloat16), w.astype(jnp.bfloat16),
                         (((1,), (0,)), ((), ())),
                         preferred_element_type=jnp.float32)


def _transform_body(v_ref, wn_ref, h_ref):
  h_ref[...] = _dot_bf(v_ref[...], wn_ref[...])


def _transform_tc(v, Wn_l):
  rowblk = pl.BlockSpec((BN, H), lambda i: (i, 0))
  return pl.pallas_call(
      _transform_body,
      grid=(N // BN,),
      in_specs=[rowblk, pl.BlockSpec((H, H), lambda i: (0, 0))],
      out_specs=rowblk,
      out_shape=jax.ShapeDtypeStruct((N, H), jnp.float32),
  )(v, Wn_l)


def _combine_body(agg_ref, wo_ref, bo_ref, v_ref):
  agg = agg_ref[0] + agg_ref[1]
  o = _dot_bf(agg, wo_ref[...]) + bo_ref[...]
  v_ref[...] = _ssp_tc(o)


def _combine_tc(AGG, Wo_l, bo_l):
  rowblk = pl.BlockSpec((BN, H), lambda i: (i, 0))
  full = lambda shape: pl.BlockSpec(shape, lambda i: tuple(0 for _ in shape))
  return pl.pallas_call(
      _combine_body,
      grid=(N // BN,),
      in_specs=[pl.BlockSpec((NC, BN, H), lambda i: (0, i, 0)),
                full((H, H)), full((1, H))],
      out_specs=rowblk,
      out_shape=jax.ShapeDtypeStruct((N, H), jnp.float32),
  )(AGG, Wo_l, bo_l)


BE = 3200


def _ew_body(dist_ref, wd_ref, bd_ref, we_ref, be_ref, ew_ref):
  e = _dot_hi(dist_ref[...], wd_ref[...]) + bd_ref[...]
  ew_ref[...] = _dot_bf(e, we_ref[...]) + be_ref[...]


def _ew_tc(dist_col, Wd, bd, We_l, be_l):
  full = lambda shape: pl.BlockSpec(shape, lambda i: tuple(0 for _ in shape))
  return pl.pallas_call(
      _ew_body,
      grid=(E // BE,),
      in_specs=[pl.BlockSpec((BE, 1), lambda i: (i, 0)),
                full((1, NG)), full((1, NG)), full((NG, H)), full((1, H))],
      out_specs=pl.BlockSpec((BE, H), lambda i: (i, 0)),
      out_shape=jax.ShapeDtypeStruct((E, H), jnp.float32),
  )(dist_col, Wd, bd, We_l, be_l)


def _readout_body(v_ref, bt_ref, w1_ref, b1_ref, w2_ref, b2_ref, u_ref):
  t1 = _ssp_tc(_dot_bf(v_ref[...], w1_ref[...]) + b1_ref[...])
  t = _dot_bf(t1, w2_ref[...]) + b2_ref[...]
  bvec = bt_ref[0, 0, :]
  oh = (bvec[:, None] == lax.broadcasted_iota(jnp.int32, (BN, B), 1))
  ub = lax.dot_general(oh.astype(jnp.float32), t, (((0,), (0,)), ((), ())),
                       precision=lax.Precision.HIGHEST,
                       preferred_element_type=jnp.float32)

  @pl.when(pl.program_id(0) == 0)
  def _():
    u_ref[...] = jnp.zeros_like(u_ref)

  u_ref[...] += ub


def _readout_tc(v, batch3, W1, b1, W2t, b2t):
  full = lambda shape: pl.BlockSpec(shape, lambda i: tuple(0 for _ in shape))
  return pl.pallas_call(
      _readout_body,
      grid=(N // BN,),
      in_specs=[pl.BlockSpec((BN, H), lambda i: (i, 0)),
                pl.BlockSpec((1, 1, BN), lambda i: (i, 0, 0)),
                full((H, H // 2)), full((1, H // 2)),
                full((H // 2, 128)), full((1, 128))],
      out_specs=full((B, 128)),
      out_shape=jax.ShapeDtypeStruct((B, 128), jnp.float32),
  )(v, batch3, W1, b1, W2t, b2t)


def kernel(z, pos, batch, edge_index, emb, W_dist, b_dist, Wn, We, be, Wo, bo,
           W1, b1, W2, b2):
  row = edge_index[0].astype(jnp.int32)
  col = edge_index[1].astype(jnp.int32)
  z32 = z.astype(jnp.int32)
  posx = pos[:, 0]
  posy = pos[:, 1]
  posz = pos[:, 2]
  zerosNH = jnp.zeros((N, H), jnp.float32)

  d2, v = _prep(posx, posy, posz, row, col, z32, emb)
  dist_col = _dist_tc(d2)

  Wd = W_dist.reshape(1, NG)
  bd = b_dist.reshape(1, NG)
  for l in range(L):
    ew = _ew_tc(dist_col, Wd, bd, We[l], be[l].reshape(1, H))
    h = _transform_tc(v, Wn[l])
    AGG = _accum(h, row, col, ew, zerosNH)
    v = _combine_tc(AGG, Wo[l], bo[l].reshape(1, H))

  batch3 = batch.astype(jnp.int32).reshape(N // BN, 1, BN)
  b1r = b1.reshape(1, H // 2)
  W2t = jnp.broadcast_to(W2, (H // 2, 128))
  b2t = jnp.broadcast_to(b2.reshape(1, 1), (1, 128))
  u128 = _readout_tc(v, batch3, W1, b1r, W2t, b2t)
  return u128[:, :1]
